# baseline (speedup 1.0000x reference)
import functools

import jax
import jax.numpy as jnp
from jax import lax
from jax.experimental import pallas as pl
from jax.experimental.pallas import tpu as pltpu
from jax.experimental.pallas import tpu_sc as plsc

NUM_CF = 13
NUM_DF = 26
EMBED_DIM = 128
VOCAB = 17
B = 16384

NSC = 13
NTC = NUM_DF - NSC
KP = 224

NC = 2
NS = 16
L = 16
NW = NC * NS
ROWS_W = B // NW
CHUNK = 256
NCHUNK = ROWS_W // CHUNK
NGROUP = CHUNK // L
TROWS = NSC * VOCAB
HALF = EMBED_DIM // 2
TSTR = HALF + 1
BSTR = EMBED_DIM + 1

MLP_BLK = 2048


def _tc_mlp_body(cf_ref, g_ref, w1_ref, b1_ref, w2_ref, b2_ref, wb_ref,
                 tab_ref, out_ref):
    cf = cf_ref[...]
    cf = jnp.where(jnp.isnan(cf), jnp.float32(0.0), cf)
    h = jnp.dot(cf, w1_ref[...], preferred_element_type=jnp.float32)
    h = jnp.maximum(h + b1_ref[...], 0.0)
    o = jnp.dot(h, w2_ref[...], preferred_element_type=jnp.float32)
    o = jnp.maximum(o + b2_ref[...], 0.0)
    o = o * wb_ref[0, 0] + wb_ref[0, 1]

    ci = lax.broadcasted_iota(jnp.int32, (NTC, KP), 1)
    ri = lax.broadcasted_iota(jnp.int32, (NTC, KP), 0)
    t_sel = (ci // VOCAB == ri).astype(jnp.float32)
    gsel = jnp.dot(g_ref[...], t_sel, preferred_element_type=jnp.float32)
    cols = lax.broadcasted_iota(jnp.int32, (MLP_BLK, KP), 1)
    onehot = (gsel.astype(jnp.int32) == cols).astype(jnp.bfloat16)
    emb = jnp.dot(onehot, tab_ref[...], preferred_element_type=jnp.float32)
    out_ref[...] = o + emb


def _tc_mlp(cf, g, w1, b1, w2, b2, wb, tab):
    return pl.pallas_call(
        _tc_mlp_body,
        grid=(B // MLP_BLK,),
        out_shape=jax.ShapeDtypeStruct((B, EMBED_DIM), jnp.float32),
        in_specs=[
            pl.BlockSpec((MLP_BLK, NUM_CF), lambda i: (i, 0)),
            pl.BlockSpec((MLP_BLK, NTC), lambda i: (i, 0)),
            pl.BlockSpec((NUM_CF, 2 * NUM_CF), lambda i: (0, 0)),
            pl.BlockSpec((1, 2 * NUM_CF), lambda i: (0, 0)),
            pl.BlockSpec((2 * NUM_CF, EMBED_DIM), lambda i: (0, 0)),
            pl.BlockSpec((1, EMBED_DIM), lambda i: (0, 0)),
            pl.BlockSpec(memory_space=pltpu.SMEM),
            pl.BlockSpec((KP, EMBED_DIM), lambda i: (0, 0)),
        ],
        out_specs=pl.BlockSpec((MLP_BLK, EMBED_DIM), lambda i: (i, 0)),
    )(cf, g, w1, b1, w2, b2, wb, tab)


def _tc_add_body(tc_ref, sc_ref, out_ref):
    out_ref[...] = tc_ref[...] + jnp.swapaxes(sc_ref[...], 0, 1)


def _tc_add(tc_part, sc_part):
    return pl.pallas_call(
        _tc_add_body,
        grid=(B // MLP_BLK,),
        out_shape=jax.ShapeDtypeStruct((B, EMBED_DIM), jnp.float32),
        in_specs=[
            pl.BlockSpec((MLP_BLK, EMBED_DIM), lambda i: (i, 0)),
            pl.BlockSpec((EMBED_DIM, MLP_BLK), lambda i: (0, i)),
        ],
        out_specs=pl.BlockSpec((MLP_BLK, EMBED_DIM), lambda i: (i, 0)),
    )(tc_part, sc_part)


def _sc_emb_body(tp_hbm, d_hbm, out_hbm, tp_v, d_v, buf0, buf1, s0, s1):
    wid = lax.axis_index("s") * NC + lax.axis_index("c")
    base = wid * ROWS_W

    bufs = (buf0, buf1)
    osems = (s0, s1)

    pltpu.sync_copy(tp_hbm, tp_v)
    pltpu.sync_copy(d_hbm.at[:, pl.ds(base, ROWS_W)], d_v)

    outcps = []
    for chunk in range(NCHUNK):
        rb = base + chunk * CHUNK

        def g_body(g, _, chunk=chunk, buf=bufs[chunk]):
            sl = chunk * CHUNK + g * L
            rows = []
            for i in range(NSC):
                dv = d_v[i, pl.ds(sl, L)]
                rows.append((dv + VOCAB * i) * TSTR)

            @plsc.parallel_loop(0, HALF, unroll=8)
            def c_body(c):
                acc_a = jnp.zeros((L,), jnp.float32)
                acc_b = jnp.zeros((L,), jnp.float32)
                for ri in rows:
                    w = plsc.bitcast(
                        plsc.load_gather(tp_v, [ri + c]), jnp.bfloat16)
                    a, b = plsc.unpack(
                        w, format=plsc.PackFormat.INTERLEAVED,
                        preferred_element_type=jnp.float32)
                    acc_a = acc_a + a
                    acc_b = acc_b + b
                buf[c, pl.ds(g * L, L)] = acc_a
                buf[c + HALF, pl.ds(g * L, L)] = acc_b

            return 0

        lax.fori_loop(0, NGROUP, g_body, 0)
        outcps.append(pltpu.async_copy(
            bufs[chunk], out_hbm.at[:, pl.ds(rb, CHUNK)], osems[chunk]))

    for cp in outcps:
        cp.wait()


_sc_emb = pl.kernel(
    _sc_emb_body,
    out_type=jax.ShapeDtypeStruct((EMBED_DIM, B), jnp.float32),
    mesh=plsc.VectorSubcoreMesh(
        core_axis_name="c", subcore_axis_name="s",
        num_cores=NC, num_subcores=NS),
    scratch_types=[
        pltpu.VMEM((TROWS * TSTR,), jnp.int32),
        pltpu.VMEM((NSC, ROWS_W), jnp.int32),
        pltpu.VMEM((EMBED_DIM, CHUNK), jnp.float32),
        pltpu.VMEM((EMBED_DIM, CHUNK), jnp.float32),
        pltpu.SemaphoreType.DMA,
        pltpu.SemaphoreType.DMA,
    ],
    compiler_params=pltpu.CompilerParams(needs_layout_passes=False),
)


def kernel(c0, c1, c2, c3, c4, c5, c6, c7, c8, c9, c10, c11, c12,
           d0, d1, d2, d3, d4, d5, d6, d7, d8, d9, d10, d11, d12,
           d13, d14, d15, d16, d17, d18, d19, d20, d21, d22, d23, d24, d25,
           W1, b1, W2, b2, Wcomb, bcomb,
           E0, E1, E2, E3, E4, E5, E6, E7, E8, E9, E10, E11, E12,
           E13, E14, E15, E16, E17, E18, E19, E20, E21, E22, E23, E24, E25):
    kw = dict(locals())
    cf = jnp.stack([kw["c%d" % i] for i in range(NUM_CF)], axis=1)
    d_sc = jnp.stack([kw["d%d" % i] for i in range(NSC)], axis=0)
    g_tc = (jnp.stack([kw["d%d" % (NSC + i)] for i in range(NTC)], axis=1)
            + VOCAB * jnp.arange(NTC, dtype=jnp.int32)[None, :]
            ).astype(jnp.float32)

    wvec = Wcomb[1:, 0]
    t_sc = jnp.concatenate(
        [kw["E%d" % i] for i in range(NSC)], axis=0)
    tb = (t_sc * jnp.repeat(wvec[:NSC], VOCAB)[:, None]).astype(jnp.bfloat16)
    pair = jnp.stack([tb[:, :HALF], tb[:, HALF:]], axis=-1)
    tp = lax.bitcast_convert_type(pair, jnp.int32)
    tp = jnp.pad(tp, ((0, 0), (0, TSTR - HALF))).reshape(-1)

    t_tc = jnp.concatenate(
        [kw["E%d" % (NSC + i)] for i in range(NTC)], axis=0)
    t_tc = (t_tc * jnp.repeat(wvec[NSC:], VOCAB)[:, None]).astype(jnp.bfloat16)
    t_tc = jnp.pad(t_tc, ((0, KP - NTC * VOCAB), (0, 0)))

    wb = jnp.stack([Wcomb[0, 0], bcomb[0]]).reshape(1, 2)
    sc_part = _sc_emb(tp, d_sc)
    tc_part = _tc_mlp(cf, g_tc, W1, b1.reshape(1, -1), W2, b2.reshape(1, -1),
                      wb, t_tc)
    return _tc_add(tc_part, sc_part)


# speedup vs baseline: 1.1119x; 1.1119x over previous
import functools

import jax
import jax.numpy as jnp
from jax import lax
from jax.experimental import pallas as pl
from jax.experimental.pallas import tpu as pltpu
from jax.experimental.pallas import tpu_sc as plsc

NUM_CF = 13
NUM_DF = 26
EMBED_DIM = 128
VOCAB = 17
B = 16384

NSC = 13
NTC = NUM_DF - NSC
KP = 224

NC = 2
NS = 16
L = 16
NW = NC * NS
ROWS_W = B // NW
CHUNK = 256
NCHUNK = ROWS_W // CHUNK
NGROUP = CHUNK // L
TROWS = NSC * VOCAB
HALF = EMBED_DIM // 2
TSTR = HALF + 1
BSTR = EMBED_DIM + 1

MLP_BLK = 2048


def _tc_mlp_body(cf_ref, g_ref, w1_ref, b1_ref, w2_ref, b2_ref, wb_ref,
                 tab_ref, out_ref):
    cf = cf_ref[...]
    cf = jnp.where(jnp.isnan(cf), jnp.float32(0.0), cf)
    h = jnp.dot(cf, w1_ref[...], preferred_element_type=jnp.float32)
    h = jnp.maximum(h + b1_ref[...], 0.0)
    o = jnp.dot(h, w2_ref[...], preferred_element_type=jnp.float32)
    o = jnp.maximum(o + b2_ref[...], 0.0)
    o = o * wb_ref[0, 0] + wb_ref[0, 1]

    ci = lax.broadcasted_iota(jnp.int32, (NTC, KP), 1)
    ri = lax.broadcasted_iota(jnp.int32, (NTC, KP), 0)
    t_sel = (ci // VOCAB == ri).astype(jnp.float32)
    gsel = jnp.dot(g_ref[...], t_sel, preferred_element_type=jnp.float32)
    cols = lax.broadcasted_iota(jnp.int32, (MLP_BLK, KP), 1)
    onehot = (gsel.astype(jnp.int32) == cols).astype(jnp.bfloat16)
    emb = jnp.dot(onehot, tab_ref[...], preferred_element_type=jnp.float32)
    out_ref[...] = o + emb


def _tc_mlp(cf, g, w1, b1, w2, b2, wb, tab):
    return pl.pallas_call(
        _tc_mlp_body,
        grid=(B // MLP_BLK,),
        out_shape=jax.ShapeDtypeStruct((B, EMBED_DIM), jnp.float32),
        in_specs=[
            pl.BlockSpec((MLP_BLK, NUM_CF), lambda i: (i, 0)),
            pl.BlockSpec((MLP_BLK, NTC), lambda i: (i, 0)),
            pl.BlockSpec((NUM_CF, 2 * NUM_CF), lambda i: (0, 0)),
            pl.BlockSpec((1, 2 * NUM_CF), lambda i: (0, 0)),
            pl.BlockSpec((2 * NUM_CF, EMBED_DIM), lambda i: (0, 0)),
            pl.BlockSpec((1, EMBED_DIM), lambda i: (0, 0)),
            pl.BlockSpec(memory_space=pltpu.SMEM),
            pl.BlockSpec((KP, EMBED_DIM), lambda i: (0, 0)),
        ],
        out_specs=pl.BlockSpec((MLP_BLK, EMBED_DIM), lambda i: (i, 0)),
    )(cf, g, w1, b1, w2, b2, wb, tab)


def _tc_add_body(tc_ref, sc_ref, out_ref):
    out_ref[...] = tc_ref[...] + jnp.swapaxes(sc_ref[...], 0, 1)


def _tc_add(tc_part, sc_part):
    return pl.pallas_call(
        _tc_add_body,
        grid=(B // MLP_BLK,),
        out_shape=jax.ShapeDtypeStruct((B, EMBED_DIM), jnp.float32),
        in_specs=[
            pl.BlockSpec((MLP_BLK, EMBED_DIM), lambda i: (i, 0)),
            pl.BlockSpec((EMBED_DIM, MLP_BLK), lambda i: (0, i)),
        ],
        out_specs=pl.BlockSpec((MLP_BLK, EMBED_DIM), lambda i: (i, 0)),
    )(tc_part, sc_part)


def _sc_emb_body(tp_hbm, d_hbm, out_hbm, tp_v, d_v, buf0, buf1, s0, s1):
    wid = lax.axis_index("s") * NC + lax.axis_index("c")
    base = wid * ROWS_W

    bufs = (buf0, buf1)
    osems = (s0, s1)

    pltpu.sync_copy(tp_hbm, tp_v)
    pltpu.sync_copy(d_hbm.at[:, pl.ds(base, ROWS_W)], d_v)

    outcps = []
    for chunk in range(NCHUNK):
        rb = base + chunk * CHUNK

        def g_body(g, _, chunk=chunk, buf=bufs[chunk]):
            sl = chunk * CHUNK + g * L
            rows = []
            for i in range(NSC):
                dv = d_v[i, pl.ds(sl, L)]
                rows.append((dv + VOCAB * i) * TSTR)

            @plsc.parallel_loop(0, HALF, unroll=4)
            def c_body(c):
                acc_a = jnp.zeros((L,), jnp.float32)
                acc_b = jnp.zeros((L,), jnp.float32)
                for ri in rows:
                    w = plsc.bitcast(
                        plsc.load_gather(tp_v, [ri + c]), jnp.bfloat16)
                    a, b = plsc.unpack(
                        w, format=plsc.PackFormat.INTERLEAVED,
                        preferred_element_type=jnp.float32)
                    acc_a = acc_a + a
                    acc_b = acc_b + b
                buf[c, pl.ds(g * L, L)] = acc_a
                buf[c + HALF, pl.ds(g * L, L)] = acc_b

            return 0

        lax.fori_loop(0, NGROUP, g_body, 0)
        outcps.append(pltpu.async_copy(
            bufs[chunk], out_hbm.at[:, pl.ds(rb, CHUNK)], osems[chunk]))

    for cp in outcps:
        cp.wait()


_sc_emb = pl.kernel(
    _sc_emb_body,
    out_type=jax.ShapeDtypeStruct((EMBED_DIM, B), jnp.float32),
    mesh=plsc.VectorSubcoreMesh(
        core_axis_name="c", subcore_axis_name="s",
        num_cores=NC, num_subcores=NS),
    scratch_types=[
        pltpu.VMEM((TROWS * TSTR,), jnp.int32),
        pltpu.VMEM((NSC, ROWS_W), jnp.int32),
        pltpu.VMEM((EMBED_DIM, CHUNK), jnp.float32),
        pltpu.VMEM((EMBED_DIM, CHUNK), jnp.float32),
        pltpu.SemaphoreType.DMA,
        pltpu.SemaphoreType.DMA,
    ],
    compiler_params=pltpu.CompilerParams(needs_layout_passes=False),
)


def kernel(c0, c1, c2, c3, c4, c5, c6, c7, c8, c9, c10, c11, c12,
           d0, d1, d2, d3, d4, d5, d6, d7, d8, d9, d10, d11, d12,
           d13, d14, d15, d16, d17, d18, d19, d20, d21, d22, d23, d24, d25,
           W1, b1, W2, b2, Wcomb, bcomb,
           E0, E1, E2, E3, E4, E5, E6, E7, E8, E9, E10, E11, E12,
           E13, E14, E15, E16, E17, E18, E19, E20, E21, E22, E23, E24, E25):
    kw = dict(locals())
    cf = jnp.stack([kw["c%d" % i] for i in range(NUM_CF)], axis=1)
    d_sc = jnp.stack([kw["d%d" % i] for i in range(NSC)], axis=0)
    g_tc = (jnp.stack([kw["d%d" % (NSC + i)] for i in range(NTC)], axis=1)
            + VOCAB * jnp.arange(NTC, dtype=jnp.int32)[None, :]
            ).astype(jnp.float32)

    wvec = Wcomb[1:, 0]
    t_sc = jnp.concatenate(
        [kw["E%d" % i] for i in range(NSC)], axis=0)
    tb = (t_sc * jnp.repeat(wvec[:NSC], VOCAB)[:, None]).astype(jnp.bfloat16)
    pair = jnp.stack([tb[:, :HALF], tb[:, HALF:]], axis=-1)
    tp = lax.bitcast_convert_type(pair, jnp.int32)
    tp = jnp.pad(tp, ((0, 0), (0, TSTR - HALF))).reshape(-1)

    t_tc = jnp.concatenate(
        [kw["E%d" % (NSC + i)] for i in range(NTC)], axis=0)
    t_tc = (t_tc * jnp.repeat(wvec[NSC:], VOCAB)[:, None]).astype(jnp.bfloat16)
    t_tc = jnp.pad(t_tc, ((0, KP - NTC * VOCAB), (0, 0)))

    wb = jnp.stack([Wcomb[0, 0], bcomb[0]]).reshape(1, 2)
    sc_part = _sc_emb(tp, d_sc)
    tc_part = _tc_mlp(cf, g_tc, W1, b1.reshape(1, -1), W2, b2.reshape(1, -1),
                      wb, t_tc)
    return _tc_add(tc_part, sc_part)
